# baseline (speedup 1.0000x reference)
import functools

import jax
import jax.numpy as jnp
from jax import lax
from jax.experimental import pallas as pl
from jax.experimental.pallas import tpu as pltpu
from jax.experimental.pallas import tpu_sc as plsc

_NC = 2
_NS = 16
_NW = _NC * _NS
_L = 16

_B = 2000


def _rsqrt(a):
    ii = lax.bitcast_convert_type(a, jnp.int32)
    ii = jnp.int32(0x5F3759DF) - lax.shift_right_logical(ii, 1)
    y = lax.bitcast_convert_type(ii, jnp.float32)
    for _ in range(3):
        y = y * (jnp.float32(1.5) - jnp.float32(0.5) * a * y * y)
    return y


def _acos(x):
    ax = jnp.abs(x)
    s = jnp.float32(1.0) - ax
    sq = s * _rsqrt(jnp.maximum(s, jnp.float32(1e-30)))
    p = jnp.float32(-0.0012624911)
    for c in (0.0066700901, -0.0170881256, 0.0308918810, -0.0501743046,
              0.0889789874, -0.2145988016, 1.5707963050):
        p = p * ax + jnp.float32(c)
    r = sq * p
    return jnp.where(x < 0, jnp.float32(3.14159265358979) - r, r)


def _make_sc_kernel(n_angles, n_atoms_p):
    per_w = n_angles // _NW
    n_blocks = per_w // _B
    per_s = n_atoms_p // _NS
    mesh = plsc.VectorSubcoreMesh(core_axis_name="c", subcore_axis_name="s")

    @functools.partial(
        pl.kernel,
        mesh=mesh,
        out_type=jax.ShapeDtypeStruct((_NW, _L), jnp.float32),
        scratch_types=[
            pltpu.VMEM_SHARED((n_atoms_p,), jnp.float32),
            pltpu.VMEM_SHARED((n_atoms_p,), jnp.float32),
            pltpu.VMEM_SHARED((n_atoms_p,), jnp.float32),
            pltpu.VMEM((_B,), jnp.int32),
            pltpu.VMEM((_B,), jnp.int32),
            pltpu.VMEM((_B,), jnp.int32),
            pltpu.VMEM((_B,), jnp.float32),
            pltpu.VMEM((_B,), jnp.float32),
            pltpu.VMEM((_B,), jnp.float32),
            pltpu.VMEM((_B,), jnp.float32),
            pltpu.VMEM((_B,), jnp.float32),
            pltpu.VMEM((_B,), jnp.float32),
            pltpu.VMEM((_B,), jnp.float32),
            pltpu.VMEM((_B,), jnp.float32),
            pltpu.VMEM((_B,), jnp.float32),
            pltpu.VMEM((_B,), jnp.float32),
            pltpu.VMEM((_B,), jnp.float32),
            pltpu.VMEM((_L,), jnp.float32),
            pltpu.SemaphoreType.DMA,
        ],
    )
    def angle_kernel(x_hbm, y_hbm, z_hbm, ai_hbm, aj_hbm, ak_hbm,
                     t0_hbm, kc_hbm, out_hbm,
                     xs_s, ys_s, zs_s,
                     ai_v, aj_v, ak_v,
                     xi_v, yi_v, zi_v, xj_v, yj_v, zj_v,
                     xk_v, yk_v, zk_v,
                     t0_v, kc_v, acc_v, sem):
        sid = lax.axis_index("s")
        wid = sid * _NC + lax.axis_index("c")

        for ch in range(per_s // _B):
            st = pl.ds(sid * per_s + ch * _B, _B)
            pltpu.sync_copy(x_hbm.at[st], xi_v)
            pltpu.sync_copy(xi_v, xs_s.at[st])
            pltpu.sync_copy(y_hbm.at[st], yi_v)
            pltpu.sync_copy(yi_v, ys_s.at[st])
            pltpu.sync_copy(z_hbm.at[st], zi_v)
            pltpu.sync_copy(zi_v, zs_s.at[st])
        plsc.subcore_barrier()

        def outer(blk, acc):
            base = wid * per_w + blk * _B
            sl = pl.ds(base, _B)
            pltpu.sync_copy(ai_hbm.at[sl], ai_v)
            pltpu.sync_copy(aj_hbm.at[sl], aj_v)
            pltpu.sync_copy(ak_hbm.at[sl], ak_v)
            cps = [
                pltpu.async_copy(x_hbm.at[ak_v], xk_v, sem),
                pltpu.async_copy(y_hbm.at[ak_v], yk_v, sem),
                pltpu.async_copy(z_hbm.at[ak_v], zk_v, sem),
                pltpu.async_copy(t0_hbm.at[sl], t0_v, sem),
                pltpu.async_copy(kc_hbm.at[sl], kc_v, sem),
            ]
            pltpu.sync_copy(xs_s.at[ai_v], xi_v)
            pltpu.sync_copy(ys_s.at[ai_v], yi_v)
            pltpu.sync_copy(zs_s.at[ai_v], zi_v)
            pltpu.sync_copy(xs_s.at[aj_v], xj_v)
            pltpu.sync_copy(ys_s.at[aj_v], yj_v)
            pltpu.sync_copy(zs_s.at[aj_v], zj_v)
            for c in cps:
                c.wait()

            def inner(g, a):
                gs = pl.ds(g * _L, _L)
                v1x = xi_v[gs] - xj_v[gs]
                v1y = yi_v[gs] - yj_v[gs]
                v1z = zi_v[gs] - zj_v[gs]
                v2x = xk_v[gs] - xj_v[gs]
                v2y = yk_v[gs] - yj_v[gs]
                v2z = zk_v[gs] - zj_v[gs]
                dot = v1x * v2x + v1y * v2y + v1z * v2z
                n1 = v1x * v1x + v1y * v1y + v1z * v1z
                n2 = v2x * v2x + v2y * v2y + v2z * v2z
                cos = dot * _rsqrt(n1 * n2)
                cos = jnp.minimum(jnp.maximum(cos, jnp.float32(-1.0)),
                                  jnp.float32(1.0))
                theta = _acos(cos)
                d = theta - t0_v[gs]
                return a + d * d * (kc_v[gs] * jnp.float32(0.5))

            return lax.fori_loop(0, _B // _L, inner, acc)

        acc = lax.fori_loop(0, n_blocks, outer,
                            jnp.zeros((_L,), jnp.float32))
        acc_v[...] = acc
        pltpu.sync_copy(acc_v, out_hbm.at[wid])

    return angle_kernel


def kernel(coords, angles, theta0, k):
    n_angles = angles.shape[0]
    n_atoms = coords.shape[0]
    n_atoms_p = ((n_atoms + _NS * _B - 1) // (_NS * _B)) * (_NS * _B)
    angles = angles.astype(jnp.int32)
    ai = angles[:, 0]
    aj = angles[:, 1]
    ak = angles[:, 2]
    cp = jnp.pad(coords, ((0, n_atoms_p - n_atoms), (0, 0)))
    x = cp[:, 0]
    y = cp[:, 1]
    z = cp[:, 2]
    partials = _make_sc_kernel(n_angles, n_atoms_p)(
        x, y, z, ai, aj, ak, theta0, k)
    return jnp.sum(partials)


# speedup vs baseline: 1.0011x; 1.0011x over previous
import functools

import jax
import jax.numpy as jnp
from jax import lax
from jax.experimental import pallas as pl
from jax.experimental.pallas import tpu as pltpu
from jax.experimental.pallas import tpu_sc as plsc

_NC = 2
_NS = 16
_NW = _NC * _NS
_L = 16

_B = 2000


def _rsqrt(a):
    ii = lax.bitcast_convert_type(a, jnp.int32)
    ii = jnp.int32(0x5F3759DF) - lax.shift_right_logical(ii, 1)
    y = lax.bitcast_convert_type(ii, jnp.float32)
    for _ in range(3):
        y = y * (jnp.float32(1.5) - jnp.float32(0.5) * a * y * y)
    return y


def _acos(x):
    ax = jnp.abs(x)
    s = jnp.float32(1.0) - ax
    sq = s * _rsqrt(jnp.maximum(s, jnp.float32(1e-30)))
    p = jnp.float32(-0.0012624911)
    for c in (0.0066700901, -0.0170881256, 0.0308918810, -0.0501743046,
              0.0889789874, -0.2145988016, 1.5707963050):
        p = p * ax + jnp.float32(c)
    r = sq * p
    return jnp.where(x < 0, jnp.float32(3.14159265358979) - r, r)


def _make_sc_kernel(n_angles, n_atoms_p):
    per_w = n_angles // _NW
    n_blocks = per_w // _B
    per_s = n_atoms_p // _NS
    mesh = plsc.VectorSubcoreMesh(core_axis_name="c", subcore_axis_name="s")

    @functools.partial(
        pl.kernel,
        mesh=mesh,
        out_type=jax.ShapeDtypeStruct((_NW, _L), jnp.float32),
        scratch_types=[
            pltpu.VMEM_SHARED((n_atoms_p,), jnp.float32),
            pltpu.VMEM_SHARED((n_atoms_p,), jnp.float32),
            pltpu.VMEM_SHARED((n_atoms_p,), jnp.float32),
            pltpu.VMEM((_B,), jnp.int32),
            pltpu.VMEM((_B,), jnp.int32),
            pltpu.VMEM((_B,), jnp.int32),
            pltpu.VMEM((_B,), jnp.float32),
            pltpu.VMEM((_B,), jnp.float32),
            pltpu.VMEM((_B,), jnp.float32),
            pltpu.VMEM((_B,), jnp.float32),
            pltpu.VMEM((_B,), jnp.float32),
            pltpu.VMEM((_B,), jnp.float32),
            pltpu.VMEM((_B,), jnp.float32),
            pltpu.VMEM((_B,), jnp.float32),
            pltpu.VMEM((_B,), jnp.float32),
            pltpu.VMEM((_B,), jnp.float32),
            pltpu.VMEM((_B,), jnp.float32),
            pltpu.VMEM((_L,), jnp.float32),
            pltpu.SemaphoreType.DMA,
            pltpu.SemaphoreType.DMA,
        ],
    )
    def angle_kernel(x_hbm, y_hbm, z_hbm, ai_hbm, aj_hbm, ak_hbm,
                     t0_hbm, kc_hbm, out_hbm,
                     xs_s, ys_s, zs_s,
                     ai_v, aj_v, ak_v,
                     xi_v, yi_v, zi_v, xj_v, yj_v, zj_v,
                     xk_v, yk_v, zk_v,
                     t0_v, kc_v, acc_v, sem, sem2):
        sid = lax.axis_index("s")
        wid = sid * _NC + lax.axis_index("c")

        for ch in range(per_s // _B):
            st = pl.ds(sid * per_s + ch * _B, _B)
            pltpu.sync_copy(x_hbm.at[st], xi_v)
            pltpu.sync_copy(xi_v, xs_s.at[st])
            pltpu.sync_copy(y_hbm.at[st], yi_v)
            pltpu.sync_copy(yi_v, ys_s.at[st])
            pltpu.sync_copy(z_hbm.at[st], zi_v)
            pltpu.sync_copy(zi_v, zs_s.at[st])
        plsc.subcore_barrier()

        def outer(blk, acc):
            base = wid * per_w + blk * _B
            sl = pl.ds(base, _B)
            pltpu.sync_copy(ai_hbm.at[sl], ai_v)
            pltpu.sync_copy(aj_hbm.at[sl], aj_v)
            pltpu.sync_copy(ak_hbm.at[sl], ak_v)
            cps = [
                pltpu.async_copy(x_hbm.at[ak_v], xk_v, sem),
                pltpu.async_copy(y_hbm.at[ak_v], yk_v, sem),
                pltpu.async_copy(z_hbm.at[ak_v], zk_v, sem),
                pltpu.async_copy(t0_hbm.at[sl], t0_v, sem),
                pltpu.async_copy(kc_hbm.at[sl], kc_v, sem),
            ]
            pltpu.async_copy(xs_s.at[ai_v], xi_v, sem2).wait()
            pltpu.async_copy(ys_s.at[ai_v], yi_v, sem2).wait()
            pltpu.async_copy(zs_s.at[ai_v], zi_v, sem2).wait()
            pltpu.async_copy(xs_s.at[aj_v], xj_v, sem2).wait()
            pltpu.async_copy(ys_s.at[aj_v], yj_v, sem2).wait()
            pltpu.async_copy(zs_s.at[aj_v], zj_v, sem2).wait()
            for c in cps:
                c.wait()

            def inner(g, a):
                gs = pl.ds(g * _L, _L)
                v1x = xi_v[gs] - xj_v[gs]
                v1y = yi_v[gs] - yj_v[gs]
                v1z = zi_v[gs] - zj_v[gs]
                v2x = xk_v[gs] - xj_v[gs]
                v2y = yk_v[gs] - yj_v[gs]
                v2z = zk_v[gs] - zj_v[gs]
                dot = v1x * v2x + v1y * v2y + v1z * v2z
                n1 = v1x * v1x + v1y * v1y + v1z * v1z
                n2 = v2x * v2x + v2y * v2y + v2z * v2z
                cos = dot * _rsqrt(n1 * n2)
                cos = jnp.minimum(jnp.maximum(cos, jnp.float32(-1.0)),
                                  jnp.float32(1.0))
                theta = _acos(cos)
                d = theta - t0_v[gs]
                return a + d * d * (kc_v[gs] * jnp.float32(0.5))

            return lax.fori_loop(0, _B // _L, inner, acc)

        acc = lax.fori_loop(0, n_blocks, outer,
                            jnp.zeros((_L,), jnp.float32))
        acc_v[...] = acc
        pltpu.sync_copy(acc_v, out_hbm.at[wid])

    return angle_kernel


def kernel(coords, angles, theta0, k):
    n_angles = angles.shape[0]
    n_atoms = coords.shape[0]
    n_atoms_p = ((n_atoms + _NS * _B - 1) // (_NS * _B)) * (_NS * _B)
    angles = angles.astype(jnp.int32)
    ai = angles[:, 0]
    aj = angles[:, 1]
    ak = angles[:, 2]
    cp = jnp.pad(coords, ((0, n_atoms_p - n_atoms), (0, 0)))
    x = cp[:, 0]
    y = cp[:, 1]
    z = cp[:, 2]
    partials = _make_sc_kernel(n_angles, n_atoms_p)(
        x, y, z, ai, aj, ak, theta0, k)
    return jnp.sum(partials)


# speedup vs baseline: 1.3111x; 1.3097x over previous
import functools

import jax
import jax.numpy as jnp
from jax import lax
from jax.experimental import pallas as pl
from jax.experimental.pallas import tpu as pltpu
from jax.experimental.pallas import tpu_sc as plsc

_NC = 2
_NS = 16
_NW = _NC * _NS
_L = 16

_B = 5000
_CS = 2000


def _rsqrt(a):
    ii = lax.bitcast_convert_type(a, jnp.int32)
    ii = jnp.int32(0x5F3759DF) - lax.shift_right_logical(ii, 1)
    y = lax.bitcast_convert_type(ii, jnp.float32)
    for _ in range(3):
        y = y * (jnp.float32(1.5) - jnp.float32(0.5) * a * y * y)
    return y


def _acos(x):
    ax = jnp.abs(x)
    s = jnp.float32(1.0) - ax
    sq = s * _rsqrt(jnp.maximum(s, jnp.float32(1e-30)))
    p = jnp.float32(-0.0012624911)
    for c in (0.0066700901, -0.0170881256, 0.0308918810, -0.0501743046,
              0.0889789874, -0.2145988016, 1.5707963050):
        p = p * ax + jnp.float32(c)
    r = sq * p
    return jnp.where(x < 0, jnp.float32(3.14159265358979) - r, r)


def _make_sc_kernel(n_angles, n_atoms_p):
    per_w = n_angles // _NW
    n_blocks = per_w // _B
    per_s = n_atoms_p // _NS
    mesh = plsc.VectorSubcoreMesh(core_axis_name="c", subcore_axis_name="s")

    @functools.partial(
        pl.kernel,
        mesh=mesh,
        out_type=jax.ShapeDtypeStruct((_NW, _L), jnp.float32),
        scratch_types=[
            pltpu.VMEM_SHARED((n_atoms_p,), jnp.float32),
            pltpu.VMEM_SHARED((n_atoms_p,), jnp.float32),
            pltpu.VMEM_SHARED((n_atoms_p,), jnp.float32),
            pltpu.VMEM((_B,), jnp.int32),
            pltpu.VMEM((_B,), jnp.int32),
            pltpu.VMEM((_B,), jnp.int32),
            pltpu.VMEM((_B,), jnp.float32),
            pltpu.VMEM((_B,), jnp.float32),
            pltpu.VMEM((_B,), jnp.float32),
            pltpu.VMEM((_B,), jnp.float32),
            pltpu.VMEM((_B,), jnp.float32),
            pltpu.VMEM((_B,), jnp.float32),
            pltpu.VMEM((_B,), jnp.float32),
            pltpu.VMEM((_B,), jnp.float32),
            pltpu.VMEM((_B,), jnp.float32),
            pltpu.VMEM((_B,), jnp.float32),
            pltpu.VMEM((_B,), jnp.float32),
            pltpu.VMEM((_L,), jnp.float32),
            pltpu.SemaphoreType.DMA,
            pltpu.SemaphoreType.DMA,
        ],
    )
    def angle_kernel(x_hbm, y_hbm, z_hbm, ai_hbm, aj_hbm, ak_hbm,
                     t0_hbm, kc_hbm, out_hbm,
                     xs_s, ys_s, zs_s,
                     ai_v, aj_v, ak_v,
                     xi_v, yi_v, zi_v, xj_v, yj_v, zj_v,
                     xk_v, yk_v, zk_v,
                     t0_v, kc_v, acc_v, sem, sem2):
        sid = lax.axis_index("s")
        wid = sid * _NC + lax.axis_index("c")

        for ch in range(per_s // _CS):
            st = pl.ds(sid * per_s + ch * _CS, _CS)
            cb = pl.ds(0, _CS)
            pltpu.sync_copy(x_hbm.at[st], xi_v.at[cb])
            pltpu.sync_copy(xi_v.at[cb], xs_s.at[st])
            pltpu.sync_copy(y_hbm.at[st], yi_v.at[cb])
            pltpu.sync_copy(yi_v.at[cb], ys_s.at[st])
            pltpu.sync_copy(z_hbm.at[st], zi_v.at[cb])
            pltpu.sync_copy(zi_v.at[cb], zs_s.at[st])
        plsc.subcore_barrier()

        def outer(blk, acc):
            base = wid * per_w + blk * _B
            sl = pl.ds(base, _B)
            pltpu.sync_copy(ai_hbm.at[sl], ai_v)
            pltpu.sync_copy(aj_hbm.at[sl], aj_v)
            pltpu.sync_copy(ak_hbm.at[sl], ak_v)
            cps = [
                pltpu.async_copy(t0_hbm.at[sl], t0_v, sem),
                pltpu.async_copy(kc_hbm.at[sl], kc_v, sem),
            ]
            pltpu.async_copy(xs_s.at[ai_v], xi_v, sem2).wait()
            pltpu.async_copy(ys_s.at[ai_v], yi_v, sem2).wait()
            pltpu.async_copy(zs_s.at[ai_v], zi_v, sem2).wait()
            pltpu.async_copy(xs_s.at[aj_v], xj_v, sem2).wait()
            pltpu.async_copy(ys_s.at[aj_v], yj_v, sem2).wait()
            pltpu.async_copy(zs_s.at[aj_v], zj_v, sem2).wait()
            pltpu.async_copy(xs_s.at[ak_v], xk_v, sem2).wait()
            pltpu.async_copy(ys_s.at[ak_v], yk_v, sem2).wait()
            pltpu.async_copy(zs_s.at[ak_v], zk_v, sem2).wait()
            for c in cps:
                c.wait()

            def inner(g, a):
                gs = pl.ds(g * _L, _L)
                v1x = xi_v[gs] - xj_v[gs]
                v1y = yi_v[gs] - yj_v[gs]
                v1z = zi_v[gs] - zj_v[gs]
                v2x = xk_v[gs] - xj_v[gs]
                v2y = yk_v[gs] - yj_v[gs]
                v2z = zk_v[gs] - zj_v[gs]
                dot = v1x * v2x + v1y * v2y + v1z * v2z
                n1 = v1x * v1x + v1y * v1y + v1z * v1z
                n2 = v2x * v2x + v2y * v2y + v2z * v2z
                cos = dot * _rsqrt(n1 * n2)
                cos = jnp.minimum(jnp.maximum(cos, jnp.float32(-1.0)),
                                  jnp.float32(1.0))
                theta = _acos(cos)
                d = theta - t0_v[gs]
                return a + d * d * (kc_v[gs] * jnp.float32(0.5))

            return lax.fori_loop(0, _B // _L, inner, acc)

        acc = lax.fori_loop(0, n_blocks, outer,
                            jnp.zeros((_L,), jnp.float32))
        acc_v[...] = acc
        pltpu.sync_copy(acc_v, out_hbm.at[wid])

    return angle_kernel


def kernel(coords, angles, theta0, k):
    n_angles = angles.shape[0]
    n_atoms = coords.shape[0]
    n_atoms_p = ((n_atoms + _NS * _CS - 1) // (_NS * _CS)) * (_NS * _CS)
    angles = angles.astype(jnp.int32)
    ai = angles[:, 0]
    aj = angles[:, 1]
    ak = angles[:, 2]
    cp = jnp.pad(coords, ((0, n_atoms_p - n_atoms), (0, 0)))
    x = cp[:, 0]
    y = cp[:, 1]
    z = cp[:, 2]
    partials = _make_sc_kernel(n_angles, n_atoms_p)(
        x, y, z, ai, aj, ak, theta0, k)
    return jnp.sum(partials)
